# baseline (speedup 1.0000x reference)
import functools

import jax
import jax.numpy as jnp
from jax import lax
from jax.experimental import pallas as pl
from jax.experimental.pallas import tpu as pltpu
from jax.experimental.pallas import tpu_sc as plsc

K_NEURONS = 10000
FEAT = 784
BATCH = 4096

BB = 512
KB = 2048
K_PAD = 10240


def _bmu_kernel(x_ref, w_ref, idx_out, min_scr, idx_scr):
    k = pl.program_id(1)
    nk = pl.num_programs(1)

    x = x_ref[...]
    w = w_ref[...]
    x_sq = jnp.sum(x * x, axis=1, keepdims=True)
    w_sq = jnp.sum(w * w, axis=1)[None, :]
    col_row = lax.broadcasted_iota(jnp.int32, (1, KB), 1) + k * KB
    w_sq = jnp.where(col_row < K_NEURONS, w_sq, jnp.inf)
    dot2 = lax.dot_general(x * 2.0, w, (((1,), (1,)), ((), ())),
                           preferred_element_type=jnp.float32)
    s = x_sq - dot2 + w_sq

    m = jnp.min(s, axis=1, keepdims=True)
    cols = lax.broadcasted_iota(jnp.int32, (BB, KB), 1) + k * KB
    idx = jnp.min(jnp.where(s == m, cols, jnp.int32(2**30)),
                  axis=1, keepdims=True)

    @pl.when(k == 0)
    def _():
        min_scr[...] = m
        idx_scr[...] = idx

    @pl.when(k > 0)
    def _():
        better = m < min_scr[...]
        min_scr[...] = jnp.where(better, m, min_scr[...])
        idx_scr[...] = jnp.where(better, idx, idx_scr[...])

    @pl.when(k == nk - 1)
    def _():
        idx_out[...] = idx_scr[...]


def _bmu_indices(x, w_padded):
    grid = (BATCH // BB, K_PAD // KB)
    return pl.pallas_call(
        _bmu_kernel,
        grid=grid,
        in_specs=[
            pl.BlockSpec((BB, FEAT), lambda b, k: (b, 0)),
            pl.BlockSpec((KB, FEAT), lambda b, k: (k, 0)),
        ],
        out_specs=pl.BlockSpec((BB, 1), lambda b, k: (b, 0)),
        out_shape=jax.ShapeDtypeStruct((BATCH, 1), jnp.int32),
        scratch_shapes=[
            pltpu.VMEM((BB, 1), jnp.float32),
            pltpu.VMEM((BB, 1), jnp.int32),
        ],
        compiler_params=pltpu.CompilerParams(
            dimension_semantics=("parallel", "arbitrary")),
    )(x, w_padded)


def _make_sc_gather():
    info = plsc.get_sparse_core_info()
    nc, ns = info.num_cores, info.num_subcores
    nw = nc * ns
    b_per_w = BATCH // nw
    mesh = plsc.VectorSubcoreMesh(core_axis_name="c", subcore_axis_name="s")

    @functools.partial(
        pl.kernel, mesh=mesh,
        compiler_params=pltpu.CompilerParams(use_tc_tiling_on_sc=False),
        out_type=jax.ShapeDtypeStruct((BATCH, FEAT), jnp.float32),
        scratch_types=[
            pltpu.VMEM((b_per_w,), jnp.int32),
            pltpu.VMEM((b_per_w, FEAT), jnp.float32),
            pltpu.SemaphoreType.DMA,
        ],
    )
    def gather(table_hbm, idx_hbm, out_hbm, idx_v, rows_v, sem):
        wid = lax.axis_index("s") * nc + lax.axis_index("c")
        base = wid * b_per_w
        pltpu.sync_copy(idx_hbm.at[pl.ds(base, b_per_w)], idx_v)
        pltpu.async_copy(table_hbm.at[idx_v], rows_v, sem).wait()
        pltpu.sync_copy(rows_v, out_hbm.at[pl.ds(base, b_per_w)])

    return gather


_sc_gather = None


def kernel(inputs, weights):
    global _sc_gather
    if _sc_gather is None:
        _sc_gather = _make_sc_gather()
    x = jnp.reshape(inputs, (BATCH, FEAT))
    w_padded = jnp.pad(weights, ((0, K_PAD - K_NEURONS), (0, 0)))
    idx = _bmu_indices(x, w_padded)
    return _sc_gather(weights, jnp.reshape(idx, (BATCH,)))


# speedup vs baseline: 1.4826x; 1.4826x over previous
import functools

import jax
import jax.numpy as jnp
from jax import lax
from jax.experimental import pallas as pl
from jax.experimental.pallas import tpu as pltpu
from jax.experimental.pallas import tpu_sc as plsc

K_NEURONS = 10000
FEAT = 784
BATCH = 4096

BB = 512
KB = 2000


def _bmu_kernel(x_ref, w_ref, idx_out, min_scr, idx_scr):
    k = pl.program_id(1)
    nk = pl.num_programs(1)

    x = x_ref[...]
    w = w_ref[...]
    x_sq = jnp.sum(x * x, axis=1, keepdims=True)
    w_sq = jnp.sum(w * w, axis=1)[None, :]
    dot2 = lax.dot_general(x * 2.0, w, (((1,), (1,)), ((), ())),
                           preferred_element_type=jnp.float32)
    s = x_sq - dot2 + w_sq

    m = jnp.min(s, axis=1, keepdims=True)
    cols = lax.broadcasted_iota(jnp.int32, (BB, KB), 1) + k * KB
    idx = jnp.min(jnp.where(s == m, cols, jnp.int32(2**30)),
                  axis=1, keepdims=True)

    @pl.when(k == 0)
    def _():
        min_scr[...] = m
        idx_scr[...] = idx

    @pl.when(k > 0)
    def _():
        better = m < min_scr[...]
        min_scr[...] = jnp.where(better, m, min_scr[...])
        idx_scr[...] = jnp.where(better, idx, idx_scr[...])

    @pl.when(k == nk - 1)
    def _():
        idx_out[...] = idx_scr[...]


def _bmu_indices(x, w):
    grid = (BATCH // BB, K_NEURONS // KB)
    return pl.pallas_call(
        _bmu_kernel,
        grid=grid,
        in_specs=[
            pl.BlockSpec((BB, FEAT), lambda b, k: (b, 0)),
            pl.BlockSpec((KB, FEAT), lambda b, k: (k, 0)),
        ],
        out_specs=pl.BlockSpec((BB, 1), lambda b, k: (b, 0)),
        out_shape=jax.ShapeDtypeStruct((BATCH, 1), jnp.int32),
        scratch_shapes=[
            pltpu.VMEM((BB, 1), jnp.float32),
            pltpu.VMEM((BB, 1), jnp.int32),
        ],
        compiler_params=pltpu.CompilerParams(
            dimension_semantics=("parallel", "arbitrary")),
    )(x, w)


def _make_sc_gather():
    info = plsc.get_sparse_core_info()
    nc, ns = info.num_cores, info.num_subcores
    nw = nc * ns
    b_per_w = BATCH // nw
    mesh = plsc.VectorSubcoreMesh(core_axis_name="c", subcore_axis_name="s")

    @functools.partial(
        pl.kernel, mesh=mesh,
        compiler_params=pltpu.CompilerParams(use_tc_tiling_on_sc=False),
        out_type=jax.ShapeDtypeStruct((BATCH, FEAT), jnp.float32),
        scratch_types=[
            pltpu.VMEM((b_per_w,), jnp.int32),
            pltpu.VMEM((b_per_w, FEAT), jnp.float32),
            pltpu.SemaphoreType.DMA,
        ],
    )
    def gather(table_hbm, idx_hbm, out_hbm, idx_v, rows_v, sem):
        wid = lax.axis_index("s") * nc + lax.axis_index("c")
        base = wid * b_per_w
        pltpu.sync_copy(idx_hbm.at[pl.ds(base, b_per_w)], idx_v)
        pltpu.async_copy(table_hbm.at[idx_v], rows_v, sem).wait()
        pltpu.sync_copy(rows_v, out_hbm.at[pl.ds(base, b_per_w)])

    return gather


_sc_gather = None


def kernel(inputs, weights):
    global _sc_gather
    if _sc_gather is None:
        _sc_gather = _make_sc_gather()
    x = jnp.reshape(inputs, (BATCH, FEAT))
    idx = _bmu_indices(x, weights)
    return _sc_gather(weights, jnp.reshape(idx, (BATCH,)))
